# Initial kernel scaffold; baseline (speedup 1.0000x reference)
import jax
import jax.numpy as jnp
from jax.experimental import pallas as pl


def kernel(pred_motion, pred_type_logits, pred_attributes, gt_motion, gt_attributes, gt_type):
    raise NotImplementedError("write your pallas kernel here")


# speedup vs baseline: 7.1416x; 7.1416x over previous
import jax
import jax.numpy as jnp
from jax import lax
from jax.experimental import pallas as pl
from jax.experimental.pallas import tpu as pltpu

_N = 1500
_NP = 1536
_M = 128
_D_MOTION = 13
_N_TYPES = 10
_N_ATTRS = 8
_F = 32
_CLIP_LO = 1e-7
_CLIP_HI = 1.0 - 1e-7


def _loss_body(pf_ref, pct_ref, gm_ref, ga_ref, gty_ref, out_ref,
               cost_ref, oh_ref):
    f32 = jnp.float32
    inf = f32(jnp.inf)
    col = lax.broadcasted_iota(jnp.int32, (1, _NP), 1)

    d0 = pct_ref[0:1, :] - gm_ref[:, 0:1]
    d1 = pct_ref[1:2, :] - gm_ref[:, 1:2]
    d2 = pct_ref[2:3, :] - gm_ref[:, 2:3]
    cost_ref[...] = (d0 * d0 + d1 * d1) + d2 * d2

    used0 = jnp.where(col < _N, f32(0.0), inf)

    def step(g, carry):
        used, ecorr = carry
        c = cost_ref[pl.ds(g, 1), :] + used
        mn = jnp.min(c)
        p = jnp.min(jnp.where(c <= mn, col, jnp.int32(2147483647)))
        oh_ref[pl.ds(g, 1), :] = jnp.where(col == p, f32(1.0), f32(0.0))
        used = jnp.where(col == p, inf, used)
        pe = jnp.clip(pf_ref[pl.ds(p, 1), 23:24], _CLIP_LO, _CLIP_HI)
        ecorr = ecorr + (jnp.log(1.0 - pe) - jnp.log(pe))
        return used, ecorr

    _, ecorr = lax.fori_loop(
        0, _M, step, (used0, jnp.zeros((1, 1), f32)), unroll=False)

    feats = jnp.dot(oh_ref[...], pf_ref[...], preferred_element_type=f32)
    mm = feats[:, 0:13]
    ml = feats[:, 13:23]
    ma = feats[:, 23:31]

    dmm = mm - gm_ref[...]
    motion_loss = jnp.sum(dmm * dmm) / f32(_M * _D_MOTION)

    mx = jnp.max(ml, axis=1, keepdims=True)
    lse = mx + jnp.log(jnp.sum(jnp.exp(ml - mx), axis=1, keepdims=True))
    toh = (lax.broadcasted_iota(jnp.int32, (_M, _N_TYPES), 1)
           == gty_ref[...]).astype(f32)
    type_loss = (jnp.sum(lse) - jnp.sum(ml * toh)) / f32(_M)

    mac = jnp.clip(ma, _CLIP_LO, _CLIP_HI)
    ga = ga_ref[...]
    bce = -(ga * jnp.log(mac) + (1.0 - ga) * jnp.log(1.0 - mac))
    attr_loss = jnp.sum(bce) / f32(_M * _N_ATTRS)

    pe_all = jnp.clip(pf_ref[:, 23:24], _CLIP_LO, _CLIP_HI)
    row = lax.broadcasted_iota(jnp.int32, (_NP, 1), 0)
    base = jnp.sum(jnp.where(row < _N, -jnp.log(1.0 - pe_all), f32(0.0)))
    exist_loss = (base + ecorr[0, 0]) / f32(_N)

    total = (motion_loss + 0.5 * type_loss + 0.5 * attr_loss
             + 2.0 * exist_loss)
    out_ref[0] = total
    out_ref[1] = motion_loss
    out_ref[2] = type_loss
    out_ref[3] = attr_loss
    out_ref[4] = exist_loss
    out_ref[5] = f32(0.0)
    out_ref[6] = f32(0.0)
    out_ref[7] = f32(0.0)


def kernel(pred_motion, pred_type_logits, pred_attributes, gt_motion,
           gt_attributes, gt_type):
    f32 = jnp.float32
    pf = jnp.zeros((_NP, _F), f32)
    pf = pf.at[:_N, 0:13].set(pred_motion.astype(f32))
    pf = pf.at[:_N, 13:23].set(pred_type_logits.astype(f32))
    pf = pf.at[:_N, 23:31].set(pred_attributes.astype(f32))
    pct = jnp.zeros((8, _NP), f32)
    pct = pct.at[0:3, :_N].set(pred_motion[:, :3].astype(f32).T)
    gty = gt_type.astype(jnp.int32).reshape(_M, 1)

    out = pl.pallas_call(
        _loss_body,
        out_shape=jax.ShapeDtypeStruct((8,), f32),
        out_specs=pl.BlockSpec(memory_space=pltpu.SMEM),
        scratch_shapes=[
            pltpu.VMEM((_M, _NP), f32),
            pltpu.VMEM((_M, _NP), f32),
        ],
    )(pf, pct, gt_motion.astype(f32), gt_attributes.astype(f32), gty)

    return (out[0], out[1], out[2], out[3], out[4])
